# baseline (speedup 1.0000x reference)
import jax
import jax.numpy as jnp
from jax import lax
from jax.experimental import pallas as pl
from jax.experimental.pallas import tpu as pltpu
from jax.experimental.pallas import tpu_sc as plsc

_B = 1024
_PER_B = 100
_REP = 4
_E = 64

_info = plsc.get_sparse_core_info()
_NC, _NS = _info.num_cores, _info.num_subcores
_NW = _NC * _NS
_BPW = _B // _NW


def _body(idx_hbm, table_hbm, out_hbm, idx_v, rows_v, sem):
    wid = lax.axis_index("s") * _NC + lax.axis_index("c")
    b0 = wid * _BPW
    pltpu.sync_copy(idx_hbm.at[pl.ds(b0, _BPW)], idx_v)

    def step(i, carry):
        pltpu.async_copy(table_hbm.at[idx_v.at[i]], rows_v, sem).wait()
        base = (b0 + i) * (_REP * _PER_B)
        for j in range(_REP):
            pltpu.sync_copy(rows_v, out_hbm.at[pl.ds(base + j * _PER_B, _PER_B)])
        return carry

    lax.fori_loop(0, _BPW, step, 0)


_gather = pl.kernel(
    _body,
    mesh=plsc.VectorSubcoreMesh(core_axis_name="c", subcore_axis_name="s"),
    out_type=jax.ShapeDtypeStruct((_B * _REP * _PER_B, _E), jnp.float32),
    scratch_types=[
        pltpu.VMEM((_BPW, _PER_B), jnp.int32),
        pltpu.VMEM((_PER_B, _E), jnp.float32),
        pltpu.SemaphoreType.DMA,
    ],
    compiler_params=pltpu.CompilerParams(use_tc_tiling_on_sc=False),
)


def kernel(x, table):
    idx = x.reshape(_B, _PER_B).astype(jnp.int32)
    flat = _gather(idx, table)
    return flat.reshape(_B, _REP * 20, 5, _E)


# speedup vs baseline: 1.0485x; 1.0485x over previous
import jax
import jax.numpy as jnp
from jax import lax
from jax.experimental import pallas as pl
from jax.experimental.pallas import tpu as pltpu
from jax.experimental.pallas import tpu_sc as plsc

_B = 1024
_PER_B = 100
_REP = 4
_E = 64

_info = plsc.get_sparse_core_info()
_NC, _NS = _info.num_cores, _info.num_subcores
_NW = _NC * _NS
_BPW = _B // _NW


_NBUF = 4
_NG = _BPW // _NBUF


def _body(idx_hbm, table_hbm, out_hbm, idx_v, rows_v,
          gsem0, gsem1, gsem2, gsem3, wsem):
    gsems = (gsem0, gsem1, gsem2, gsem3)
    wid = lax.axis_index("s") * _NC + lax.axis_index("c")
    b0 = wid * _BPW
    pltpu.sync_copy(idx_hbm.at[pl.ds(b0, _BPW)], idx_v)

    for s in range(_NBUF):
        pltpu.async_copy(table_hbm.at[idx_v.at[s]], rows_v.at[s], gsems[s])

    def step(g, carry):
        for s in range(_NBUF):
            i = g * _NBUF + s
            pltpu.make_async_copy(
                table_hbm.at[idx_v.at[i]], rows_v.at[s], gsems[s]).wait()
            base = (b0 + i) * (_REP * _PER_B)
            handles = [
                pltpu.async_copy(
                    rows_v.at[s],
                    out_hbm.at[pl.ds(base + j * _PER_B, _PER_B)], wsem)
                for j in range(_REP)
            ]
            for h in handles:
                h.wait()
            nxt = i + _NBUF

            @pl.when(nxt < _BPW)
            def _():
                pltpu.async_copy(
                    table_hbm.at[idx_v.at[nxt]], rows_v.at[s], gsems[s])
        return carry

    lax.fori_loop(0, _NG, step, 0)


_gather = pl.kernel(
    _body,
    mesh=plsc.VectorSubcoreMesh(core_axis_name="c", subcore_axis_name="s"),
    out_type=jax.ShapeDtypeStruct((_B * _REP * _PER_B, _E), jnp.float32),
    scratch_types=[
        pltpu.VMEM((_BPW, _PER_B), jnp.int32),
        pltpu.VMEM((_NBUF, _PER_B, _E), jnp.float32),
        pltpu.SemaphoreType.DMA,
        pltpu.SemaphoreType.DMA,
        pltpu.SemaphoreType.DMA,
        pltpu.SemaphoreType.DMA,
        pltpu.SemaphoreType.DMA,
    ],
    compiler_params=pltpu.CompilerParams(use_tc_tiling_on_sc=False),
)


def kernel(x, table):
    idx = x.reshape(_B, _PER_B).astype(jnp.int32)
    flat = _gather(idx, table)
    return flat.reshape(_B, _REP * 20, 5, _E)


# speedup vs baseline: 1.3451x; 1.2829x over previous
import jax
import jax.numpy as jnp
from jax import lax
from jax.experimental import pallas as pl
from jax.experimental.pallas import tpu as pltpu
from jax.experimental.pallas import tpu_sc as plsc

_N = 102400
_E = 64
_CH = 128
_NBUF = 5

_info = plsc.get_sparse_core_info()
_NC, _NS = _info.num_cores, _info.num_subcores
_NW = _NC * _NS
_PW = _N // _NW
_NG = _PW // _CH


def _sc_body(idx_hbm, table_hbm, out_hbm, idx_v, rows_v,
             g0, g1, g2, g3, g4, wsem):
    gsems = (g0, g1, g2, g3, g4)
    wid = lax.axis_index("s") * _NC + lax.axis_index("c")
    base = wid * _PW
    pltpu.sync_copy(idx_hbm.at[pl.ds(base, _PW)], idx_v)

    for s in range(_NBUF):
        pltpu.async_copy(
            table_hbm.at[idx_v.at[pl.ds(_CH * s, _CH)]], rows_v.at[s],
            gsems[s])

    def step(g, carry):
        for s in range(_NBUF):
            i = g * _NBUF + s
            pltpu.make_async_copy(
                table_hbm.at[idx_v.at[pl.ds(i * _CH, _CH)]], rows_v.at[s],
                gsems[s]).wait()
            pltpu.async_copy(
                rows_v.at[s], out_hbm.at[pl.ds(base + i * _CH, _CH)],
                wsem).wait()
            nxt = i + _NBUF

            @pl.when(nxt < _NG)
            def _():
                pltpu.async_copy(
                    table_hbm.at[idx_v.at[pl.ds(nxt * _CH, _CH)]],
                    rows_v.at[s], gsems[s])
        return carry

    lax.fori_loop(0, _NG // _NBUF, step, 0)


_sc = pl.kernel(
    _sc_body,
    mesh=plsc.VectorSubcoreMesh(core_axis_name="c", subcore_axis_name="s"),
    out_type=jax.ShapeDtypeStruct((_N, _E), jnp.float32),
    scratch_types=[
        pltpu.VMEM((_PW,), jnp.int32),
        pltpu.VMEM((_NBUF, _CH, _E), jnp.float32),
        pltpu.SemaphoreType.DMA,
        pltpu.SemaphoreType.DMA,
        pltpu.SemaphoreType.DMA,
        pltpu.SemaphoreType.DMA,
        pltpu.SemaphoreType.DMA,
        pltpu.SemaphoreType.DMA,
    ],
    compiler_params=pltpu.CompilerParams(use_tc_tiling_on_sc=False),
)


def _tc_body(in_ref, out_ref):
    blk = in_ref[...]
    t = jnp.transpose(blk)
    out_ref[...] = jnp.broadcast_to(t[None, None], (4, 1, _E, _CH))


_tc = pl.pallas_call(
    _tc_body,
    grid=(100, 8),
    in_specs=[pl.BlockSpec((_CH, _E), lambda r, c: (r * 8 + c, 0))],
    out_specs=pl.BlockSpec((4, 1, _E, _CH), lambda r, c: (0, r, 0, c)),
    out_shape=jax.ShapeDtypeStruct((4, 100, _E, 1024), jnp.float32),
)


def kernel(x, table):
    xt = x.transpose(1, 2, 0).reshape(-1).astype(jnp.int32)
    inter = _sc(xt, table)
    out4 = _tc(inter)
    return (out4.reshape(80, 5, _E, 1024).transpose(3, 0, 1, 2))


# speedup vs baseline: 2.3619x; 1.7559x over previous
import jax
import jax.numpy as jnp
from jax import lax
from jax.experimental import pallas as pl
from jax.experimental.pallas import tpu as pltpu
from jax.experimental.pallas import tpu_sc as plsc

_N = 102400
_E = 64
_CH = 128
_NBUF = 5

_info = plsc.get_sparse_core_info()
_NC, _NS = _info.num_cores, _info.num_subcores
_NW = _NC * _NS
_PW = _N // _NW
_NG = _PW // _CH


def _sc_body(idx_hbm, table_hbm, out_hbm, idx_v, rows_v,
             g0, g1, g2, g3, g4, wsem):
    gsems = (g0, g1, g2, g3, g4)
    wid = lax.axis_index("s") * _NC + lax.axis_index("c")
    base = wid * _PW
    pltpu.sync_copy(idx_hbm.at[pl.ds(base, _PW)], idx_v)

    for s in range(_NBUF):
        pltpu.async_copy(
            table_hbm.at[idx_v.at[pl.ds(_CH * s, _CH)]], rows_v.at[s],
            gsems[s])

    def step(g, carry):
        for s in range(_NBUF):
            i = g * _NBUF + s
            pltpu.make_async_copy(
                table_hbm.at[idx_v.at[pl.ds(i * _CH, _CH)]], rows_v.at[s],
                gsems[s]).wait()
            pltpu.async_copy(
                rows_v.at[s], out_hbm.at[pl.ds(base + i * _CH, _CH)],
                wsem).wait()
            nxt = i + _NBUF

            @pl.when(nxt < _NG)
            def _():
                pltpu.async_copy(
                    table_hbm.at[idx_v.at[pl.ds(nxt * _CH, _CH)]],
                    rows_v.at[s], gsems[s])
        return carry

    lax.fori_loop(0, _NG // _NBUF, step, 0)


_sc = pl.kernel(
    _sc_body,
    mesh=plsc.VectorSubcoreMesh(core_axis_name="c", subcore_axis_name="s"),
    out_type=jax.ShapeDtypeStruct((_N, _E), jnp.float32),
    scratch_types=[
        pltpu.VMEM((_PW,), jnp.int32),
        pltpu.VMEM((_NBUF, _CH, _E), jnp.float32),
        pltpu.SemaphoreType.DMA,
        pltpu.SemaphoreType.DMA,
        pltpu.SemaphoreType.DMA,
        pltpu.SemaphoreType.DMA,
        pltpu.SemaphoreType.DMA,
        pltpu.SemaphoreType.DMA,
    ],
    compiler_params=pltpu.CompilerParams(use_tc_tiling_on_sc=False),
)


_RP = 4


def _tc_body(in_ref, out_ref):
    for i in range(_RP):
        t = jnp.transpose(in_ref[pl.ds(i * 1024, 1024), :])
        out_ref[:, i, :, :] = jnp.broadcast_to(t[None], (4, _E, 1024))


_tc = pl.pallas_call(
    _tc_body,
    grid=(100 // _RP,),
    in_specs=[pl.BlockSpec((_RP * 1024, _E), lambda g: (g, 0))],
    out_specs=pl.BlockSpec((4, _RP, _E, 1024), lambda g: (0, g, 0, 0)),
    out_shape=jax.ShapeDtypeStruct((4, 100, _E, 1024), jnp.float32),
)


def kernel(x, table):
    xt = x.transpose(1, 2, 0).reshape(-1).astype(jnp.int32)
    inter = _sc(xt, table)
    out4 = _tc(inter)
    return (out4.reshape(80, 5, _E, 1024).transpose(3, 0, 1, 2))


# speedup vs baseline: 2.6697x; 1.1303x over previous
import jax
import jax.numpy as jnp
from jax import lax
from jax.experimental import pallas as pl
from jax.experimental.pallas import tpu as pltpu
from jax.experimental.pallas import tpu_sc as plsc

_N = 102400
_E = 64
_CH = 128
_NBUF = 5

_info = plsc.get_sparse_core_info()
_NC, _NS = _info.num_cores, _info.num_subcores
_NW = _NC * _NS
_PW = _N // _NW
_NG = _PW // _CH


def _sc_body(idx_hbm, table_hbm, out_hbm, idx_v, rows_v,
             g0, g1, g2, g3, g4, wsem):
    gsems = (g0, g1, g2, g3, g4)
    wid = lax.axis_index("s") * _NC + lax.axis_index("c")
    base = wid * _PW
    pltpu.sync_copy(idx_hbm.at[pl.ds(base, _PW)], idx_v)

    for s in range(_NBUF):
        pltpu.async_copy(
            table_hbm.at[idx_v.at[pl.ds(_CH * s, _CH)]], rows_v.at[s],
            gsems[s])

    def step(g, carry):
        for s in range(_NBUF):
            i = g * _NBUF + s
            pltpu.make_async_copy(
                table_hbm.at[idx_v.at[pl.ds(i * _CH, _CH)]], rows_v.at[s],
                gsems[s]).wait()
            pltpu.async_copy(
                rows_v.at[s], out_hbm.at[pl.ds(base + i * _CH, _CH)],
                wsem).wait()
            nxt = i + _NBUF

            @pl.when(nxt < _NG)
            def _():
                pltpu.async_copy(
                    table_hbm.at[idx_v.at[pl.ds(nxt * _CH, _CH)]],
                    rows_v.at[s], gsems[s])
        return carry

    lax.fori_loop(0, _NG // _NBUF, step, 0)


_sc = pl.kernel(
    _sc_body,
    mesh=plsc.VectorSubcoreMesh(core_axis_name="c", subcore_axis_name="s"),
    out_type=jax.ShapeDtypeStruct((_N, _E), jnp.float32),
    scratch_types=[
        pltpu.VMEM((_PW,), jnp.int32),
        pltpu.VMEM((_NBUF, _CH, _E), jnp.float32),
        pltpu.SemaphoreType.DMA,
        pltpu.SemaphoreType.DMA,
        pltpu.SemaphoreType.DMA,
        pltpu.SemaphoreType.DMA,
        pltpu.SemaphoreType.DMA,
        pltpu.SemaphoreType.DMA,
    ],
    compiler_params=pltpu.CompilerParams(use_tc_tiling_on_sc=False),
)


_RP = 4


def _tc_body(in_ref, out_ref):
    for i in range(_RP):
        sub = in_ref[pl.ds(i * 512, 512), :]
        t0 = jnp.transpose(sub[:, 0:_E])
        t1 = jnp.transpose(sub[:, _E:2 * _E])
        out_ref[:, i, :, 0:512] = jnp.broadcast_to(t0[None], (4, _E, 512))
        out_ref[:, i, :, 512:1024] = jnp.broadcast_to(t1[None], (4, _E, 512))


_tc = pl.pallas_call(
    _tc_body,
    grid=(100 // _RP,),
    in_specs=[pl.BlockSpec((_RP * 512, 128), lambda g: (g, 0))],
    out_specs=pl.BlockSpec((4, _RP, _E, 1024), lambda g: (0, g, 0, 0)),
    out_shape=jax.ShapeDtypeStruct((4, 100, _E, 1024), jnp.float32),
)


def kernel(x, table):
    xt = x.transpose(1, 2, 0).reshape(100, 1024).astype(jnp.int32)
    xt = xt.reshape(100, 2, 512).transpose(0, 2, 1).reshape(-1)
    inter = _sc(xt, table)
    out4 = _tc(inter.reshape(51200, 128))
    return (out4.reshape(80, 5, _E, 1024).transpose(3, 0, 1, 2))


# speedup vs baseline: 2.7527x; 1.0311x over previous
import jax
import jax.numpy as jnp
from jax import lax
from jax.experimental import pallas as pl
from jax.experimental.pallas import tpu as pltpu
from jax.experimental.pallas import tpu_sc as plsc

_N = 102400
_E = 64
_CH = 128
_NBUF = 5

_info = plsc.get_sparse_core_info()
_NC, _NS = _info.num_cores, _info.num_subcores
_NW = _NC * _NS
_PW = _N // _NW
_NG = _PW // _CH


def _sc_body(idx_hbm, table_hbm, out_hbm, idx_v, rows_v,
             g0, g1, g2, g3, g4, wsem):
    gsems = (g0, g1, g2, g3, g4)
    wid = lax.axis_index("s") * _NC + lax.axis_index("c")
    base = wid * _PW
    pltpu.sync_copy(idx_hbm.at[pl.ds(base, _PW)], idx_v)

    for s in range(_NBUF):
        pltpu.async_copy(
            table_hbm.at[idx_v.at[pl.ds(_CH * s, _CH)]], rows_v.at[s],
            gsems[s])

    def step(g, carry):
        for s in range(_NBUF):
            i = g * _NBUF + s
            pltpu.make_async_copy(
                table_hbm.at[idx_v.at[pl.ds(i * _CH, _CH)]], rows_v.at[s],
                gsems[s]).wait()
            pltpu.async_copy(
                rows_v.at[s], out_hbm.at[pl.ds(base + i * _CH, _CH)],
                wsem).wait()
            nxt = i + _NBUF

            @pl.when(nxt < _NG)
            def _():
                pltpu.async_copy(
                    table_hbm.at[idx_v.at[pl.ds(nxt * _CH, _CH)]],
                    rows_v.at[s], gsems[s])
        return carry

    lax.fori_loop(0, _NG // _NBUF, step, 0)


_sc = pl.kernel(
    _sc_body,
    mesh=plsc.VectorSubcoreMesh(core_axis_name="c", subcore_axis_name="s"),
    out_type=jax.ShapeDtypeStruct((_N, _E), jnp.float32),
    scratch_types=[
        pltpu.VMEM((_PW,), jnp.int32),
        pltpu.VMEM((_NBUF, _CH, _E), jnp.float32),
        pltpu.SemaphoreType.DMA,
        pltpu.SemaphoreType.DMA,
        pltpu.SemaphoreType.DMA,
        pltpu.SemaphoreType.DMA,
        pltpu.SemaphoreType.DMA,
        pltpu.SemaphoreType.DMA,
    ],
    compiler_params=pltpu.CompilerParams(use_tc_tiling_on_sc=False),
)


_RP = 10


def _tc_body(in_ref, out_ref):
    for i in range(_RP):
        sub = in_ref[pl.ds(i * 512, 512), :]
        t0 = jnp.transpose(sub[:, 0:_E])
        t1 = jnp.transpose(sub[:, _E:2 * _E])
        out_ref[:, i, :, 0:512] = jnp.broadcast_to(t0[None], (4, _E, 512))
        out_ref[:, i, :, 512:1024] = jnp.broadcast_to(t1[None], (4, _E, 512))


_tc = pl.pallas_call(
    _tc_body,
    grid=(100 // _RP,),
    in_specs=[pl.BlockSpec((_RP * 512, 128), lambda g: (g, 0))],
    out_specs=pl.BlockSpec((4, _RP, _E, 1024), lambda g: (0, g, 0, 0)),
    out_shape=jax.ShapeDtypeStruct((4, 100, _E, 1024), jnp.float32),
)


def kernel(x, table):
    xt = x.transpose(1, 2, 0).reshape(100, 1024).astype(jnp.int32)
    xt = xt.reshape(100, 2, 512).transpose(0, 2, 1).reshape(-1)
    inter = _sc(xt, table)
    out4 = _tc(inter.reshape(51200, 128))
    return (out4.reshape(80, 5, _E, 1024).transpose(3, 0, 1, 2))


# speedup vs baseline: 2.7531x; 1.0002x over previous
import jax
import jax.numpy as jnp
from jax import lax
from jax.experimental import pallas as pl
from jax.experimental.pallas import tpu as pltpu
from jax.experimental.pallas import tpu_sc as plsc

_N = 102400
_E = 64
_CH = 128
_NBUF = 5

_info = plsc.get_sparse_core_info()
_NC, _NS = _info.num_cores, _info.num_subcores
_NW = _NC * _NS
_PW = _N // _NW
_NG = _PW // _CH


def _sc_body(idx_hbm, table_hbm, out_hbm, idx_v, rows_v,
             g0, g1, g2, g3, g4, wsem):
    gsems = (g0, g1, g2, g3, g4)
    wid = lax.axis_index("s") * _NC + lax.axis_index("c")
    base = wid * _PW
    pltpu.sync_copy(idx_hbm.at[pl.ds(base, _PW)], idx_v)

    for s in range(_NBUF):
        pltpu.async_copy(
            table_hbm.at[idx_v.at[pl.ds(_CH * s, _CH)]], rows_v.at[s],
            gsems[s])

    def step(g, carry):
        for s in range(_NBUF):
            i = g * _NBUF + s
            pltpu.make_async_copy(
                table_hbm.at[idx_v.at[pl.ds(i * _CH, _CH)]], rows_v.at[s],
                gsems[s]).wait()
            pltpu.async_copy(
                rows_v.at[s], out_hbm.at[pl.ds(base + i * _CH, _CH)],
                wsem).wait()
            nxt = i + _NBUF

            @pl.when(nxt < _NG)
            def _():
                pltpu.async_copy(
                    table_hbm.at[idx_v.at[pl.ds(nxt * _CH, _CH)]],
                    rows_v.at[s], gsems[s])
        return carry

    lax.fori_loop(0, _NG // _NBUF, step, 0)


_sc = pl.kernel(
    _sc_body,
    mesh=plsc.VectorSubcoreMesh(core_axis_name="c", subcore_axis_name="s"),
    out_type=jax.ShapeDtypeStruct((_N, 2 * _E), jnp.float32),
    scratch_types=[
        pltpu.VMEM((_PW,), jnp.int32),
        pltpu.VMEM((_NBUF, _CH, 2 * _E), jnp.float32),
        pltpu.SemaphoreType.DMA,
        pltpu.SemaphoreType.DMA,
        pltpu.SemaphoreType.DMA,
        pltpu.SemaphoreType.DMA,
        pltpu.SemaphoreType.DMA,
        pltpu.SemaphoreType.DMA,
    ],
    compiler_params=pltpu.CompilerParams(use_tc_tiling_on_sc=True),
)

_RP = 10


def _tc_body(in_ref, par_ref, out_ref):
    for i in range(_RP):
        blk = in_ref[pl.ds(i * 1024, 1024), :]
        row = pl.program_id(0) * _RP + i
        p = par_ref[pl.ds(row, 1), :]
        tf = jnp.transpose(blk)
        t = jnp.where(p == 1, tf[_E:2 * _E, :], tf[0:_E, :])
        out_ref[:, i, :, :] = jnp.broadcast_to(t[None], (4, _E, 1024))


_tc = pl.pallas_call(
    _tc_body,
    grid=(100 // _RP,),
    in_specs=[
        pl.BlockSpec((_RP * 1024, 128), lambda g: (g, 0)),
        pl.BlockSpec((100, 1024), lambda g: (0, 0)),
    ],
    out_specs=pl.BlockSpec((4, _RP, _E, 1024), lambda g: (0, g, 0, 0)),
    out_shape=jax.ShapeDtypeStruct((4, 100, _E, 1024), jnp.float32),
)


def kernel(x, table):
    xt = x.transpose(1, 2, 0).reshape(-1).astype(jnp.int32)
    midx = xt >> 1
    par = (xt & 1).reshape(100, 1024)
    inter = _sc(midx, table.reshape(50000, 2 * _E))
    out4 = _tc(inter, par)
    return (out4.reshape(80, 5, _E, 1024).transpose(3, 0, 1, 2))
